# baseline (speedup 1.0000x reference)
import functools

import jax
import jax.numpy as jnp
from jax import lax
from jax.experimental import pallas as pl
from jax.experimental.pallas import tpu as pltpu
from jax.experimental.pallas import tpu_sc as plsc

S = 64
D = 9
C = 4
B = 16
N = 16384
HALVES = 2
XH = S // HALVES
CELLS = XH * S
SENT = CELLS
CNT_W = 2064
OUT_W = CELLS * D * C
OUT_PAD = OUT_W + 48
CHUNK = 4096
NCHUNKS = N // CHUNK
GROUPS = CHUNK // 16

_mesh = functools.partial(
    plsc.VectorSubcoreMesh, core_axis_name="c", subcore_axis_name="s"
)


def _sc_body(data_hbm, out_hbm, buf0_v, buf1_v, out_v, cnt_v, in_sem):
  h = lax.axis_index("c")
  b = lax.axis_index("s")

  zf = jnp.zeros((16,), jnp.float32)
  zi = jnp.zeros((16,), jnp.int32)

  def zero_out(i, carry):
    out_v[pl.ds(i * 16, 16)] = zf
    return carry

  lax.fori_loop(0, OUT_PAD // 16, zero_out, 0, unroll=8)

  def zero_cnt(i, carry):
    cnt_v[pl.ds(i * 16, 16)] = zi
    return carry

  lax.fori_loop(0, CNT_W // 16, zero_cnt, 0, unroll=8)

  bufs = (buf0_v, buf1_v)

  def chunk_copy(ci, slot):
    return pltpu.make_async_copy(
        data_hbm.at[b, pl.ds(ci * CHUNK * C, CHUNK * C)], bufs[slot], in_sem
    )

  chunk_copy(0, 0).start()

  hbase = h * CELLS
  lane4 = lax.iota(jnp.int32, 16) * C

  for ci in range(NCHUNKS):
    slot = ci % 2
    chunk_copy(ci, slot).wait()
    if ci + 1 < NCHUNKS:
      chunk_copy(ci + 1, 1 - slot).start()
    buf = bufs[slot]

    def group(g, carry):
      b0 = g * (16 * C) + lane4
      r0 = plsc.load_gather(buf, [b0])
      r1 = plsc.load_gather(buf, [b0 + 1])
      r2 = plsc.load_gather(buf, [b0 + 2])
      r3 = plsc.load_gather(buf, [b0 + 3])
      d0 = r0 * float(S)
      d1 = r1 * float(S)
      xi = d0.astype(jnp.int32)
      yi = d1.astype(jnp.int32)
      f0 = d0 - xi.astype(jnp.float32)
      f1 = d1 - yi.astype(jnp.float32)
      nz = (r0 != 0.0) | (r1 != 0.0) | (r2 != 0.0) | (r3 != 0.0)
      mine = (xi >> 5) == h
      lcell = xi * S + yi - hbase
      ceff = jnp.where(nz & mine, lcell, SENT)
      cnt = plsc.load_gather(cnt_v, [ceff])
      dup, last = plsc.scan_count(ceff)
      rank = cnt + dup - 1
      plsc.store_scatter(cnt_v, [ceff], rank + 1, mask=last)
      ok = (ceff != SENT) & (rank < D)
      rc = jnp.minimum(rank, D - 1)
      base = ceff * (D * C) + rc * C
      plsc.store_scatter(out_v, [base], f0, mask=ok)
      plsc.store_scatter(out_v, [base + 1], f1, mask=ok)
      plsc.store_scatter(out_v, [base + 2], r2, mask=ok)
      plsc.store_scatter(out_v, [base + 3], r3, mask=ok)
      return carry

    lax.fori_loop(0, GROUPS, group, 0)

  pltpu.sync_copy(out_v.at[pl.ds(0, OUT_W)], out_hbm.at[b, h])


@jax.jit
def kernel(data):
  launch = pl.kernel(
      _sc_body,
      out_type=jax.ShapeDtypeStruct((B, HALVES, OUT_W), jnp.float32),
      mesh=_mesh(),
      scratch_types=[
          pltpu.VMEM((CHUNK * C,), jnp.float32),
          pltpu.VMEM((CHUNK * C,), jnp.float32),
          pltpu.VMEM((OUT_PAD,), jnp.float32),
          pltpu.VMEM((CNT_W,), jnp.int32),
          pltpu.SemaphoreType.DMA,
      ],
      compiler_params=pltpu.CompilerParams(needs_layout_passes=False),
  )
  out = launch(data.reshape(B, N * C))
  return out.reshape(B, S, S, D * C)


# speedup vs baseline: 1.6290x; 1.6290x over previous
import functools

import jax
import jax.numpy as jnp
from jax import lax
from jax.experimental import pallas as pl
from jax.experimental.pallas import tpu as pltpu
from jax.experimental.pallas import tpu_sc as plsc

S = 64
D = 9
C = 4
B = 16
N = 16384
HALVES = 2
XH = S // HALVES
CELLS = XH * S
SENT = CELLS
CNT_W = 2064
PLANES = D * C
PB = 128
NB = N // PB
JCHUNK = 16
NCHUNKS = NB // JCHUNK

_mesh = functools.partial(
    plsc.VectorSubcoreMesh, core_axis_name="c", subcore_axis_name="s"
)


def _sc_body(data_hbm, out_hbm, buf_v, out_v, cnt_v, in_sem):
  h = lax.axis_index("c")
  b = lax.axis_index("s")

  zf = jnp.zeros((16,), jnp.float32)
  zi = jnp.zeros((16,), jnp.int32)

  def zero_out(i, carry):
    k = i >> 7
    p0 = (i & 127) * 16
    out_v[k, pl.ds(p0, 16)] = zf
    return carry

  lax.fori_loop(0, PLANES * 128, zero_out, 0, unroll=8)

  def zero_cnt(i, carry):
    cnt_v[pl.ds(i * 16, 16)] = zi
    return carry

  lax.fori_loop(0, CNT_W // 16, zero_cnt, 0, unroll=8)

  def chunk_copy(ci, slot):
    return pltpu.make_async_copy(
        data_hbm.at[b, pl.ds(ci * JCHUNK, JCHUNK), :, :], buf_v.at[slot], in_sem
    )

  chunk_copy(0, 0).start()

  for ci in range(NCHUNKS):
    slot = ci % 2
    chunk_copy(ci, slot).wait()
    if ci + 1 < NCHUNKS:
      chunk_copy(ci + 1, 1 - slot).start()
    buf = buf_v.at[slot]

    def block(jj, carry):
      for pp in range(PB // 16):
        p0 = pp * 16
        r0 = buf[jj, 0, pl.ds(p0, 16)]
        r1 = buf[jj, 1, pl.ds(p0, 16)]
        r2 = buf[jj, 2, pl.ds(p0, 16)]
        r3 = buf[jj, 3, pl.ds(p0, 16)]
        d0 = r0 * float(S)
        d1 = r1 * float(S)
        xi = d0.astype(jnp.int32)
        yi = d1.astype(jnp.int32)
        f0 = d0 - xi.astype(jnp.float32)
        f1 = d1 - yi.astype(jnp.float32)
        nz = (r0 != 0.0) | (r1 != 0.0) | (r2 != 0.0) | (r3 != 0.0)
        mine = (xi >> 5) == h
        xl = xi & (XH - 1)
        lcell = xl * S + yi
        ceff = jnp.where(nz & mine, lcell, SENT)
        cnt = plsc.load_gather(cnt_v, [ceff])
        dup, last = plsc.scan_count(ceff)
        rank = cnt + dup - 1
        plsc.store_scatter(cnt_v, [ceff], rank + 1, mask=last)
        ok = (ceff != SENT) & (rank < D)
        rc4 = jnp.minimum(rank, D - 1) * C
        plsc.store_scatter(out_v, [rc4, lcell], f0, mask=ok)
        plsc.store_scatter(out_v, [rc4 + 1, lcell], f1, mask=ok)
        plsc.store_scatter(out_v, [rc4 + 2, lcell], r2, mask=ok)
        plsc.store_scatter(out_v, [rc4 + 3, lcell], r3, mask=ok)
      return carry

    lax.fori_loop(0, JCHUNK, block, 0)

  pltpu.sync_copy(out_v, out_hbm.at[b, :, pl.ds(h * CELLS, CELLS)])


@jax.jit
def kernel(data):
  launch = pl.kernel(
      _sc_body,
      out_type=jax.ShapeDtypeStruct((B, PLANES, S * S), jnp.float32),
      mesh=_mesh(),
      scratch_types=[
          pltpu.VMEM((2, JCHUNK, C, PB), jnp.float32),
          pltpu.VMEM((PLANES, XH * S), jnp.float32),
          pltpu.VMEM((CNT_W,), jnp.int32),
          pltpu.SemaphoreType.DMA,
      ],
      compiler_params=pltpu.CompilerParams(needs_layout_passes=False),
  )
  dt = data.reshape(B, NB, PB, C).transpose(0, 1, 3, 2)
  planar = launch(dt).reshape(B, PLANES, S, S)
  return jnp.transpose(planar, (0, 2, 3, 1))
